# baseline (speedup 1.0000x reference)
import functools

import jax
import jax.numpy as jnp
from jax import lax
from jax.experimental import pallas as pl
from jax.experimental.pallas import tpu as pltpu
from jax.experimental.pallas import tpu_sc as plsc

B = 256
N = 32768
C = 16
L = 16
NC = 2
NS = 16
NW = NC * NS
RPW = B // NW
K = 4096
NCHUNK = N // K
HIST = 256


def _body(values_hbm, labels_hbm, out_hbm, cnt_hbm,
          vbuf, lbuf, acc, cnt_acc, outb, cntb,
          sem_v0, sem_v1, sem_l0, sem_l1):
  wid = lax.axis_index("s") * NC + lax.axis_index("c")
  row_base = wid * RPW
  sem_v = (sem_v0, sem_v1)
  sem_l = (sem_l0, sem_l1)

  zeros = jnp.zeros((L,), jnp.float32)
  ones = jnp.ones((L,), jnp.float32)
  iota = lax.broadcasted_iota(jnp.int32, (L,), 0)
  lane_base = iota * C

  def _zero(i, _):
    acc[pl.ds(i * L, L)] = zeros
    return 0
  lax.fori_loop(0, (RPW * HIST) // L, _zero, 0)

  def _zero_cnt(i, _):
    cnt_acc[pl.ds(i * L, L)] = zeros
    return 0
  lax.fori_loop(0, HIST // L, _zero_cnt, 0)

  def start(g):
    b = g % 2
    cv = pltpu.async_copy(
        values_hbm.at[pl.ds(row_base, RPW), pl.ds(g * K, K)],
        vbuf.at[b], sem_v[b])
    cl = pltpu.async_copy(
        labels_hbm.at[pl.ds(g * K, K)], lbuf.at[b], sem_l[b])
    return cv, cl

  inflight = start(0)
  for g in range(NCHUNK):
    b = g % 2
    cv, cl = inflight
    cv.wait()
    cl.wait()
    if g + 1 < NCHUNK:
      inflight = start(g + 1)

    def _group(j, _):
      lab = lbuf[b, pl.ds(j * L, L)]
      idx0 = lab + lane_base
      for r in range(RPW):
        v = vbuf[b, r, pl.ds(j * L, L)]
        plsc.addupdate_scatter(acc, [idx0 + (r * HIST)], v)
      return 0
    lax.fori_loop(0, K // L, _group, 0)

    @pl.when(wid == 0)
    def _():
      def _cnt(j, _):
        lab = lbuf[b, pl.ds(j * L, L)]
        plsc.addupdate_scatter(cnt_acc, [lab + lane_base], ones)
        return 0
      lax.fori_loop(0, K // L, _cnt, 0)

  for r in range(RPW):
    def _red(l, s):
      return s + acc[pl.ds(r * HIST + l * L, L)]
    outb[r, :] = lax.fori_loop(0, L, _red, zeros)
  pltpu.sync_copy(outb, out_hbm.at[pl.ds(row_base, RPW), :])

  @pl.when(wid == 0)
  def _():
    def _redc(l, s):
      return s + cnt_acc[pl.ds(l * L, L)]
    cntb[...] = lax.fori_loop(0, L, _redc, zeros)
    pltpu.sync_copy(cntb, cnt_hbm)


@jax.jit
def _pooled_sums(values, labels):
  return pl.kernel(
      _body,
      out_type=(
          jax.ShapeDtypeStruct((B, C), jnp.float32),
          jax.ShapeDtypeStruct((C,), jnp.float32),
      ),
      mesh=plsc.VectorSubcoreMesh(
          core_axis_name="c", subcore_axis_name="s",
          num_cores=NC, num_subcores=NS),
      compiler_params=pltpu.CompilerParams(needs_layout_passes=False),
      scratch_types=[
          pltpu.VMEM((2, RPW, K), jnp.float32),
          pltpu.VMEM((2, K), jnp.int32),
          pltpu.VMEM((RPW * HIST,), jnp.float32),
          pltpu.VMEM((HIST,), jnp.float32),
          pltpu.VMEM((RPW, C), jnp.float32),
          pltpu.VMEM((C,), jnp.float32),
          pltpu.SemaphoreType.DMA,
          pltpu.SemaphoreType.DMA,
          pltpu.SemaphoreType.DMA,
          pltpu.SemaphoreType.DMA,
      ],
  )(values, labels)


def kernel(values, labels):
  sums, counts = _pooled_sums(values, labels)
  present = counts > 0.5
  pos = jnp.cumsum(present.astype(jnp.int32)) - 1
  clabels = (jnp.zeros((C,), jnp.int32)
             .at[jnp.where(present, pos, C)]
             .set(jnp.arange(C, dtype=jnp.int32), mode="drop"))
  return jnp.take(sums, clabels, axis=1)


# speedup vs baseline: 1.5375x; 1.5375x over previous
import functools

import jax
import jax.numpy as jnp
from jax import lax
from jax.experimental import pallas as pl
from jax.experimental.pallas import tpu as pltpu
from jax.experimental.pallas import tpu_sc as plsc

B = 256
N = 32768
C = 16
L = 16
NC = 2
NS = 16
NW = NC * NS
RPW = B // NW
K = 4096
NCHUNK = N // K
HIST = 256


def _body(values_hbm, labels_hbm, out_hbm, cnt_hbm,
          vbuf, lbuf, acc, cnt_acc, outb, cntb,
          sem_v0, sem_v1, sem_l0, sem_l1):
  wid = lax.axis_index("s") * NC + lax.axis_index("c")
  row_base = wid * RPW
  sem_v = (sem_v0, sem_v1)
  sem_l = (sem_l0, sem_l1)

  zeros = jnp.zeros((L,), jnp.float32)
  ones = jnp.ones((L,), jnp.float32)
  iota = lax.broadcasted_iota(jnp.int32, (L,), 0)
  lane_base = iota * C

  def _zero(i, _):
    acc[pl.ds(i * L, L)] = zeros
    return 0
  lax.fori_loop(0, (RPW * HIST) // L, _zero, 0)

  def _zero_cnt(i, _):
    cnt_acc[pl.ds(i * L, L)] = zeros
    return 0
  lax.fori_loop(0, HIST // L, _zero_cnt, 0)

  def start(g):
    b = g % 2
    cv = pltpu.async_copy(
        values_hbm.at[pl.ds(row_base, RPW), pl.ds(g * K, K)],
        vbuf.at[b], sem_v[b])
    cl = pltpu.async_copy(
        labels_hbm.at[pl.ds(g * K, K)], lbuf.at[b], sem_l[b])
    return cv, cl

  inflight = start(0)
  for g in range(NCHUNK):
    b = g % 2
    cv, cl = inflight
    cv.wait()
    cl.wait()
    if g + 1 < NCHUNK:
      inflight = start(g + 1)

    def _group(j, _):
      lab = lbuf[b, pl.ds(j * L, L)]
      idx0 = lab + lane_base
      vs = [vbuf[b, r, pl.ds(j * L, L)] for r in range(RPW)]
      idxs = [idx0 + (r * HIST) for r in range(RPW)]
      for r in range(RPW):
        plsc.addupdate_scatter(acc, [idxs[r]], vs[r])
      return 0
    lax.fori_loop(0, K // L, _group, 0)

    @pl.when(wid == 0)
    def _():
      def _cnt(j, _):
        lab = lbuf[b, pl.ds(j * L, L)]
        plsc.addupdate_scatter(cnt_acc, [lab + lane_base], ones)
        return 0
      lax.fori_loop(0, K // L, _cnt, 0)

  for r in range(RPW):
    def _red(l, s):
      return s + acc[pl.ds(r * HIST + l * L, L)]
    outb[r, :] = lax.fori_loop(0, L, _red, zeros)
  pltpu.sync_copy(outb, out_hbm.at[pl.ds(row_base, RPW), :])

  @pl.when(wid == 0)
  def _():
    def _redc(l, s):
      return s + cnt_acc[pl.ds(l * L, L)]
    cntb[...] = lax.fori_loop(0, L, _redc, zeros)
    pltpu.sync_copy(cntb, cnt_hbm)


@jax.jit
def _pooled_sums(values, labels):
  return pl.kernel(
      _body,
      out_type=(
          jax.ShapeDtypeStruct((B, C), jnp.float32),
          jax.ShapeDtypeStruct((C,), jnp.float32),
      ),
      mesh=plsc.VectorSubcoreMesh(
          core_axis_name="c", subcore_axis_name="s",
          num_cores=NC, num_subcores=NS),
      compiler_params=pltpu.CompilerParams(needs_layout_passes=False),
      scratch_types=[
          pltpu.VMEM((2, RPW, K), jnp.float32),
          pltpu.VMEM((2, K), jnp.int32),
          pltpu.VMEM((RPW * HIST,), jnp.float32),
          pltpu.VMEM((HIST,), jnp.float32),
          pltpu.VMEM((RPW, C), jnp.float32),
          pltpu.VMEM((C,), jnp.float32),
          pltpu.SemaphoreType.DMA,
          pltpu.SemaphoreType.DMA,
          pltpu.SemaphoreType.DMA,
          pltpu.SemaphoreType.DMA,
      ],
  )(values, labels)


def kernel(values, labels):
  sums, counts = _pooled_sums(values, labels)
  present = counts > 0.5
  pos = jnp.cumsum(present.astype(jnp.int32)) - 1
  clabels = (jnp.zeros((C,), jnp.int32)
             .at[jnp.where(present, pos, C)]
             .set(jnp.arange(C, dtype=jnp.int32), mode="drop"))
  return jnp.take(sums, clabels, axis=1)


# speedup vs baseline: 1.8802x; 1.2229x over previous
import functools

import jax
import jax.numpy as jnp
from jax import lax
from jax.experimental import pallas as pl
from jax.experimental.pallas import tpu as pltpu
from jax.experimental.pallas import tpu_sc as plsc

B = 256
N = 32768
C = 16
L = 16
NC = 2
NS = 16
NW = NC * NS
RPW = B // NW
K = 4096
NCHUNK = N // K
HIST = 256


def _body(values_hbm, labels_hbm, out_hbm, cnt_hbm,
          vbuf, lbuf, acc, cnt_acc, outb, cntb,
          sem_v0, sem_v1, sem_l0, sem_l1):
  wid = lax.axis_index("s") * NC + lax.axis_index("c")
  row_base = wid * RPW
  sem_v = (sem_v0, sem_v1)
  sem_l = (sem_l0, sem_l1)

  zeros = jnp.zeros((L,), jnp.float32)
  ones = jnp.ones((L,), jnp.float32)
  iota = lax.broadcasted_iota(jnp.int32, (L,), 0)
  lane_base = iota * C

  def _zero(i, _):
    acc[pl.ds(i * L, L)] = zeros
    return 0
  lax.fori_loop(0, (RPW * HIST) // L, _zero, 0)

  def _zero_cnt(i, _):
    cnt_acc[pl.ds(i * L, L)] = zeros
    return 0
  lax.fori_loop(0, HIST // L, _zero_cnt, 0)

  def start(g):
    b = g % 2
    cv = pltpu.async_copy(
        values_hbm.at[pl.ds(row_base, RPW), pl.ds(g * K, K)],
        vbuf.at[b], sem_v[b])
    cl = pltpu.async_copy(
        labels_hbm.at[pl.ds(g * K, K)], lbuf.at[b], sem_l[b])
    return cv, cl

  inflight = start(0)
  for g in range(NCHUNK):
    b = g % 2
    cv, cl = inflight
    cv.wait()
    cl.wait()
    if g + 1 < NCHUNK:
      inflight = start(g + 1)

    @plsc.parallel_loop(0, K // L, unroll=4)
    def _group(j):
      lab = lbuf[b, pl.ds(j * L, L)]
      idx0 = lab + lane_base
      vs = [vbuf[b, r, pl.ds(j * L, L)] for r in range(RPW)]
      idxs = [idx0 + (r * HIST) for r in range(RPW)]
      for r in range(RPW):
        plsc.addupdate_scatter(acc, [idxs[r]], vs[r])

    def _cnt(jj, _):
      lab = lbuf[b, pl.ds((wid * (K // L // NW) + jj) * L, L)]
      plsc.addupdate_scatter(cnt_acc, [lab + lane_base], ones)
      return 0
    lax.fori_loop(0, K // L // NW, _cnt, 0)

  for r in range(RPW):
    def _red(l, s):
      return s + acc[pl.ds(r * HIST + l * L, L)]
    outb[r, :] = lax.fori_loop(0, L, _red, zeros)
  pltpu.sync_copy(outb, out_hbm.at[pl.ds(row_base, RPW), :])

  def _redc(l, s):
    return s + cnt_acc[pl.ds(l * L, L)]
  cntb[...] = lax.fori_loop(0, L, _redc, zeros)
  pltpu.sync_copy(cntb, cnt_hbm.at[wid])


@jax.jit
def _pooled_sums(values, labels):
  return pl.kernel(
      _body,
      out_type=(
          jax.ShapeDtypeStruct((B, C), jnp.float32),
          jax.ShapeDtypeStruct((NW, C), jnp.float32),
      ),
      mesh=plsc.VectorSubcoreMesh(
          core_axis_name="c", subcore_axis_name="s",
          num_cores=NC, num_subcores=NS),
      compiler_params=pltpu.CompilerParams(needs_layout_passes=False),
      scratch_types=[
          pltpu.VMEM((2, RPW, K), jnp.float32),
          pltpu.VMEM((2, K), jnp.int32),
          pltpu.VMEM((RPW * HIST,), jnp.float32),
          pltpu.VMEM((HIST,), jnp.float32),
          pltpu.VMEM((RPW, C), jnp.float32),
          pltpu.VMEM((C,), jnp.float32),
          pltpu.SemaphoreType.DMA,
          pltpu.SemaphoreType.DMA,
          pltpu.SemaphoreType.DMA,
          pltpu.SemaphoreType.DMA,
      ],
  )(values, labels)


def kernel(values, labels):
  sums, count_parts = _pooled_sums(values, labels)
  present = jnp.sum(count_parts, axis=0) > 0.5
  pos = jnp.cumsum(present.astype(jnp.int32)) - 1
  clabels = (jnp.zeros((C,), jnp.int32)
             .at[jnp.where(present, pos, C)]
             .set(jnp.arange(C, dtype=jnp.int32), mode="drop"))
  return jnp.take(sums, clabels, axis=1)


# speedup vs baseline: 1.9190x; 1.0207x over previous
import functools

import jax
import jax.numpy as jnp
from jax import lax
from jax.experimental import pallas as pl
from jax.experimental.pallas import tpu as pltpu
from jax.experimental.pallas import tpu_sc as plsc

B = 256
N = 32768
C = 16
L = 16
NC = 2
NS = 16
NW = NC * NS
RPW = B // NW
K = 4096
NCHUNK = N // K
HIST = 256
GPC = K // L


def _body(values_hbm, labels_hbm, out_hbm,
          vbuf, lbuf, acc, cnt_acc, outb, cntb, cnt_all, rsbuf, clb_v,
          cnt_sh, sem_v0, sem_v1, sem_l0, sem_l1):
  cid = lax.axis_index("c")
  sid = lax.axis_index("s")
  wid = sid * NC + cid
  row_base = wid * RPW
  sem_v = (sem_v0, sem_v1)
  sem_l = (sem_l0, sem_l1)

  zeros = jnp.zeros((L,), jnp.float32)
  ones = jnp.ones((L,), jnp.float32)
  iota = lax.broadcasted_iota(jnp.int32, (L,), 0)
  lane_base = iota * C

  def _zero(i, _):
    acc[pl.ds(i * L, L)] = zeros
    return 0
  lax.fori_loop(0, (RPW * HIST) // L, _zero, 0)

  def _zero_cnt(i, _):
    cnt_acc[pl.ds(i * L, L)] = zeros
    return 0
  lax.fori_loop(0, HIST // L, _zero_cnt, 0)

  def start(g):
    b = g % 2
    cv = pltpu.async_copy(
        values_hbm.at[pl.ds(row_base, RPW), pl.ds(g * K, K)],
        vbuf.at[b], sem_v[b])
    cl = pltpu.async_copy(
        labels_hbm.at[pl.ds(g * K, K)], lbuf.at[b], sem_l[b])
    return cv, cl

  inflight = start(0)
  for g in range(NCHUNK):
    b = g % 2
    cv, cl = inflight
    cv.wait()
    cl.wait()
    if g + 1 < NCHUNK:
      inflight = start(g + 1)

    @plsc.parallel_loop(0, GPC, unroll=4)
    def _group(j):
      lab = lbuf[b, pl.ds(j * L, L)]
      idx0 = lab + lane_base
      vs = [vbuf[b, r, pl.ds(j * L, L)] for r in range(RPW)]
      idxs = [idx0 + (r * HIST) for r in range(RPW)]
      for r in range(RPW):
        plsc.addupdate_scatter(acc, [idxs[r]], vs[r])

    def _cnt(jj, _):
      lab = lbuf[b, pl.ds((sid * (GPC // NS) + jj) * L, L)]
      plsc.addupdate_scatter(cnt_acc, [lab + lane_base], ones)
      return 0
    lax.fori_loop(0, GPC // NS, _cnt, 0)

  def _redc(l, s):
    return s + cnt_acc[pl.ds(l * L, L)]
  cntb[...] = lax.fori_loop(0, L, _redc, zeros)
  pltpu.sync_copy(cntb, cnt_sh.at[sid])
  plsc.subcore_barrier()
  pltpu.sync_copy(cnt_sh, cnt_all)

  def _redall(t, s):
    return s + cnt_all[t, :]
  counts = lax.fori_loop(0, NS, _redall, zeros)

  present = counts > 0.5
  pos = plsc.cumsum(present.astype(jnp.int32)) - 1
  clb_v[...] = jnp.zeros((C,), jnp.int32)
  plsc.store_scatter(clb_v, [pos], iota, mask=present)
  clab = clb_v[...]

  for r in range(RPW):
    def _red(l, s):
      return s + acc[pl.ds(r * HIST + l * L, L)]
    rsbuf[...] = lax.fori_loop(0, L, _red, zeros)
    outb[r, :] = plsc.load_gather(rsbuf, [clab])
  pltpu.sync_copy(outb, out_hbm.at[pl.ds(row_base, RPW), :])


@jax.jit
def _pooled_sums(values, labels):
  return pl.kernel(
      _body,
      out_type=jax.ShapeDtypeStruct((B, C), jnp.float32),
      mesh=plsc.VectorSubcoreMesh(
          core_axis_name="c", subcore_axis_name="s",
          num_cores=NC, num_subcores=NS),
      compiler_params=pltpu.CompilerParams(needs_layout_passes=False),
      scratch_types=[
          pltpu.VMEM((2, RPW, K), jnp.float32),
          pltpu.VMEM((2, K), jnp.int32),
          pltpu.VMEM((RPW * HIST,), jnp.float32),
          pltpu.VMEM((HIST,), jnp.float32),
          pltpu.VMEM((RPW, C), jnp.float32),
          pltpu.VMEM((C,), jnp.float32),
          pltpu.VMEM((NS, C), jnp.float32),
          pltpu.VMEM((C,), jnp.float32),
          pltpu.VMEM((C,), jnp.int32),
          pltpu.VMEM_SHARED((NS, C), jnp.float32),
          pltpu.SemaphoreType.DMA,
          pltpu.SemaphoreType.DMA,
          pltpu.SemaphoreType.DMA,
          pltpu.SemaphoreType.DMA,
      ],
  )(values, labels)


def kernel(values, labels):
  return _pooled_sums(values, labels)
